# baseline (speedup 1.0000x reference)
import functools

import jax
import jax.numpy as jnp
from jax import lax
from jax.experimental import pallas as pl
from jax.experimental.pallas import tpu as pltpu
from jax.experimental.pallas import tpu_sc as plsc

B = 16384
D = 64
N_REL = 64
N_AUX = 3
KG_LAMBDA = 1e-05

NC = 2
NS = 16
NW = NC * NS
BPW = B // NW
CH = 128
NCH = BPW // CH
AUXW = 16

BT = 512
G = B // BT


def _sc_gather_body(h_hbm, p_hbm, n_hbm, perm_hbm, r_hbm, tab_hbm, aux_hbm,
                    rel_hbm, out_all, idx_v, rows_v, sem):
    c = lax.axis_index("c")
    s = lax.axis_index("s")
    wid = s * NC + c
    base = wid * BPW
    for t, (src, tab) in enumerate(
            ((h_hbm, tab_hbm), (p_hbm, tab_hbm), (n_hbm, tab_hbm),
             (perm_hbm, aux_hbm), (r_hbm, rel_hbm))):
        for j in range(NCH):
            pltpu.sync_copy(src.at[pl.ds(base + j * CH, CH)], idx_v.at[j])
        handles = [
            pltpu.async_copy(tab.at[idx_v.at[j]],
                             rows_v.at[pl.ds(j * CH, CH)], sem)
            for j in range(NCH)
        ]
        for hd in handles:
            hd.wait()
        pltpu.sync_copy(rows_v, out_all.at[pl.ds(base, BPW), t])


@functools.lru_cache(maxsize=1)
def _sc_gather_kernel():
    return functools.partial(
        pl.kernel,
        mesh=plsc.VectorSubcoreMesh(core_axis_name="c", subcore_axis_name="s"),
        out_type=jax.ShapeDtypeStruct((B, 5, D), jnp.float32),
        scratch_types=[
            pltpu.VMEM((NCH, CH), jnp.int32),
            pltpu.VMEM((BPW, D), jnp.float32),
            pltpu.SemaphoreType.DMA,
        ],
        compiler_params=pltpu.CompilerParams(use_tc_tiling_on_sc=False),
    )(_sc_gather_body)


def _tc_body(sref, all_ref, rf_ref, m_ref, awt_ref, ab_ref, out_ref):
    g = pl.program_id(0)
    start = sref[0, g]
    cnt = sref[1, g]
    awt = awt_ref[...]
    ab = ab_ref[...]
    e_all = all_ref[...]
    auxp = e_all[:, 3, :]

    def fuse(e, a):
        v = jnp.tanh(jnp.dot(a, awt, preferred_element_type=jnp.float32) + ab)
        return e * (1.0 + v)

    he = fuse(e_all[:, 0, :], auxp[:, 0:N_AUX])
    pe = fuse(e_all[:, 1, :], auxp[:, N_AUX:2 * N_AUX])
    ne = fuse(e_all[:, 2, :], auxp[:, 2 * N_AUX:3 * N_AUX])
    rr = rf_ref[...]
    e3 = jnp.concatenate([he, pe, ne], axis=0)
    rr3 = jnp.concatenate([rr, rr, rr], axis=0)

    def body(w, acc):
        k = start + w
        mk = m_ref[k]
        kf = lax.convert_element_type(k, jnp.float32)
        em = jnp.where(rr3 == kf, e3, 0.0)
        return acc + jnp.dot(em, mk, preferred_element_type=jnp.float32)

    z = jnp.zeros((3 * BT, D), jnp.float32)
    acc3 = lax.fori_loop(0, cnt, body, z)
    acc_h = acc3[0:BT, :]
    acc_p = acc3[BT:2 * BT, :]
    acc_n = acc3[2 * BT:3 * BT, :]

    re = e_all[:, 4, :]
    dp = acc_h + re - acc_p
    dn = acc_h + re - acc_n
    pos = jnp.sum(dp * dp, axis=1, keepdims=True)
    neg = jnp.sum(dn * dn, axis=1, keepdims=True)
    x = neg - pos
    nls = jnp.maximum(-x, 0.0) + jnp.log(1.0 + jnp.exp(-jnp.abs(x)))
    rh2 = jnp.sum(acc_h * acc_h, axis=1, keepdims=True)
    re2 = jnp.sum(re * re, axis=1, keepdims=True)
    rp2 = jnp.sum(acc_p * acc_p, axis=1, keepdims=True)
    rn2 = jnp.sum(acc_n * acc_n, axis=1, keepdims=True)
    zz = jnp.zeros((BT, 3), jnp.float32)
    cat = jnp.concatenate([nls, rh2, re2, rp2, rn2, zz], axis=1)
    out_ref[...] = jnp.sum(cat, axis=0, keepdims=True).reshape(1, 1, 8)


def _tc_call(sinfo, e_all, rf, m, awt, ab):
    grid_spec = pltpu.PrefetchScalarGridSpec(
        num_scalar_prefetch=1,
        grid=(G,),
        in_specs=[
            pl.BlockSpec((BT, 5, D), lambda g, s: (g, 0, 0)),
            pl.BlockSpec((BT, 1), lambda g, s: (g, 0)),
            pl.BlockSpec((N_REL, D, D), lambda g, s: (0, 0, 0)),
            pl.BlockSpec((N_AUX, D), lambda g, s: (0, 0)),
            pl.BlockSpec((1, D), lambda g, s: (0, 0)),
        ],
        out_specs=pl.BlockSpec((1, 1, 8), lambda g, s: (g, 0, 0)),
    )
    return pl.pallas_call(
        _tc_body,
        grid_spec=grid_spec,
        out_shape=jax.ShapeDtypeStruct((G, 1, 8), jnp.float32),
    )(sinfo, e_all, rf, m, awt, ab)


def kernel(h, r, pos_t, neg_t, h_aux, pos_t_aux, neg_t_aux,
           entity_user_embed, relation_embed, trans_M, aux_W, aux_b):
    h = h.astype(jnp.int32)
    r = r.astype(jnp.int32)
    pos_t = pos_t.astype(jnp.int32)
    neg_t = neg_t.astype(jnp.int32)
    perm = jnp.argsort(r).astype(jnp.int32)
    r_s = jnp.take(r, perm)
    h_s = jnp.take(h, perm)
    p_s = jnp.take(pos_t, perm)
    n_s = jnp.take(neg_t, perm)
    aux_pack = jnp.concatenate(
        [h_aux, pos_t_aux, neg_t_aux,
         jnp.zeros((B, D - 3 * N_AUX), jnp.float32)], axis=1)
    e_all = _sc_gather_kernel()(
        h_s, p_s, n_s, perm, r_s, entity_user_embed, aux_pack,
        relation_embed)
    starts = r_s[0::BT]
    cnts = r_s[BT - 1::BT] - starts + 1
    sinfo = jnp.stack([starts, cnts]).astype(jnp.int32)
    rf = r_s.astype(jnp.float32).reshape(B, 1)
    res = _tc_call(sinfo, e_all, rf,
                   trans_M, aux_W.T, aux_b.reshape(1, D))
    p = jnp.sum(res, axis=(0, 1))
    kg_loss = p[0] / B
    l2_loss = (p[1] + p[2] + p[3] + p[4]) / (2.0 * B)
    return kg_loss + KG_LAMBDA * l2_loss


# speedup vs baseline: 1.0780x; 1.0780x over previous
import functools

import jax
import jax.numpy as jnp
from jax import lax
from jax.experimental import pallas as pl
from jax.experimental.pallas import tpu as pltpu
from jax.experimental.pallas import tpu_sc as plsc

B = 16384
D = 64
N_REL = 64
N_AUX = 3
KG_LAMBDA = 1e-05

NC = 2
NS = 16
NW = NC * NS
BPW = B // NW
CH = 128
NCH = BPW // CH
AUXW = 16

BT = 512
G = B // BT


def _sc_gather_body(h_hbm, p_hbm, n_hbm, perm_hbm, r_hbm, tab_hbm, aux_hbm,
                    rel_hbm, out_h, out_p, out_n, out_a, out_r,
                    idx_v, rows_v, aux_v, sem):
    c = lax.axis_index("c")
    s = lax.axis_index("s")
    wid = s * NC + c
    base = wid * BPW
    for src, tab, dst in ((h_hbm, tab_hbm, out_h), (p_hbm, tab_hbm, out_p),
                          (n_hbm, tab_hbm, out_n), (r_hbm, rel_hbm, out_r)):
        for j in range(NCH):
            pltpu.sync_copy(src.at[pl.ds(base + j * CH, CH)], idx_v.at[j])
        handles = [
            pltpu.async_copy(tab.at[idx_v.at[j]],
                             rows_v.at[pl.ds(j * CH, CH)], sem)
            for j in range(NCH)
        ]
        for hd in handles:
            hd.wait()
        pltpu.sync_copy(rows_v, dst.at[pl.ds(base, BPW)])
    for j in range(NCH):
        pltpu.sync_copy(perm_hbm.at[pl.ds(base + j * CH, CH)], idx_v.at[j])
    handles = [
        pltpu.async_copy(aux_hbm.at[idx_v.at[j]],
                         aux_v.at[pl.ds(j * CH, CH)], sem)
        for j in range(NCH)
    ]
    for hd in handles:
        hd.wait()
    pltpu.sync_copy(aux_v, out_a.at[pl.ds(base, BPW)])


@functools.lru_cache(maxsize=1)
def _sc_gather_kernel():
    return functools.partial(
        pl.kernel,
        mesh=plsc.VectorSubcoreMesh(core_axis_name="c", subcore_axis_name="s"),
        out_type=(
            jax.ShapeDtypeStruct((B, D), jnp.float32),
            jax.ShapeDtypeStruct((B, D), jnp.float32),
            jax.ShapeDtypeStruct((B, D), jnp.float32),
            jax.ShapeDtypeStruct((B, AUXW), jnp.float32),
            jax.ShapeDtypeStruct((B, D), jnp.float32),
        ),
        scratch_types=[
            pltpu.VMEM((NCH, CH), jnp.int32),
            pltpu.VMEM((BPW, D), jnp.float32),
            pltpu.VMEM((BPW, AUXW), jnp.float32),
            pltpu.SemaphoreType.DMA,
        ],
        compiler_params=pltpu.CompilerParams(use_tc_tiling_on_sc=False),
    )(_sc_gather_body)


def _tc_body(sref, he_ref, pe_ref, ne_ref, auxp_ref, relg_ref, rf_ref,
             m_ref, awt_ref, ab_ref, out_ref):
    g = pl.program_id(0)
    start = sref[0, g]
    cnt = sref[1, g]
    awt = awt_ref[...]
    ab = ab_ref[...]
    auxp = auxp_ref[...]

    def fuse(e_ref, a):
        v = jnp.tanh(jnp.dot(a, awt, preferred_element_type=jnp.float32) + ab)
        return e_ref[...] * (1.0 + v)

    he = fuse(he_ref, auxp[:, 0:N_AUX])
    pe = fuse(pe_ref, auxp[:, N_AUX:2 * N_AUX])
    ne = fuse(ne_ref, auxp[:, 2 * N_AUX:3 * N_AUX])
    rr = rf_ref[...]
    e3 = jnp.concatenate([he, pe, ne], axis=0)
    rr3 = jnp.concatenate([rr, rr, rr], axis=0)

    def body(w, acc):
        k = start + w
        mk = m_ref[k]
        kf = lax.convert_element_type(k, jnp.float32)
        em = jnp.where(rr3 == kf, e3, 0.0)
        return acc + jnp.dot(em, mk, preferred_element_type=jnp.float32)

    z = jnp.zeros((3 * BT, D), jnp.float32)
    acc3 = lax.fori_loop(0, cnt, body, z)
    acc_h = acc3[0:BT, :]
    acc_p = acc3[BT:2 * BT, :]
    acc_n = acc3[2 * BT:3 * BT, :]

    re = relg_ref[...]
    dp = acc_h + re - acc_p
    dn = acc_h + re - acc_n
    pos = jnp.sum(dp * dp, axis=1, keepdims=True)
    neg = jnp.sum(dn * dn, axis=1, keepdims=True)
    x = neg - pos
    nls = jnp.maximum(-x, 0.0) + jnp.log(1.0 + jnp.exp(-jnp.abs(x)))
    rh2 = jnp.sum(acc_h * acc_h, axis=1, keepdims=True)
    re2 = jnp.sum(re * re, axis=1, keepdims=True)
    rp2 = jnp.sum(acc_p * acc_p, axis=1, keepdims=True)
    rn2 = jnp.sum(acc_n * acc_n, axis=1, keepdims=True)
    zz = jnp.zeros((BT, 3), jnp.float32)
    cat = jnp.concatenate([nls, rh2, re2, rp2, rn2, zz], axis=1)
    out_ref[...] = jnp.sum(cat, axis=0, keepdims=True).reshape(1, 1, 8)


def _tc_call(sinfo, he, pe, ne, auxp, relg, rf, m, awt, ab):
    grid_spec = pltpu.PrefetchScalarGridSpec(
        num_scalar_prefetch=1,
        grid=(G,),
        in_specs=[
            pl.BlockSpec((BT, D), lambda g, s: (g, 0)),
            pl.BlockSpec((BT, D), lambda g, s: (g, 0)),
            pl.BlockSpec((BT, D), lambda g, s: (g, 0)),
            pl.BlockSpec((BT, AUXW), lambda g, s: (g, 0)),
            pl.BlockSpec((BT, D), lambda g, s: (g, 0)),
            pl.BlockSpec((BT, 1), lambda g, s: (g, 0)),
            pl.BlockSpec((N_REL, D, D), lambda g, s: (0, 0, 0)),
            pl.BlockSpec((N_AUX, D), lambda g, s: (0, 0)),
            pl.BlockSpec((1, D), lambda g, s: (0, 0)),
        ],
        out_specs=pl.BlockSpec((1, 1, 8), lambda g, s: (g, 0, 0)),
    )
    return pl.pallas_call(
        _tc_body,
        grid_spec=grid_spec,
        out_shape=jax.ShapeDtypeStruct((G, 1, 8), jnp.float32),
    )(sinfo, he, pe, ne, auxp, relg, rf, m, awt, ab)


def kernel(h, r, pos_t, neg_t, h_aux, pos_t_aux, neg_t_aux,
           entity_user_embed, relation_embed, trans_M, aux_W, aux_b):
    h = h.astype(jnp.int32)
    r = r.astype(jnp.int32)
    pos_t = pos_t.astype(jnp.int32)
    neg_t = neg_t.astype(jnp.int32)
    perm = jnp.argsort(r).astype(jnp.int32)
    r_s = jnp.take(r, perm)
    h_s = jnp.take(h, perm)
    p_s = jnp.take(pos_t, perm)
    n_s = jnp.take(neg_t, perm)
    aux_pack = jnp.concatenate(
        [h_aux, pos_t_aux, neg_t_aux,
         jnp.zeros((B, AUXW - 3 * N_AUX), jnp.float32)], axis=1)
    he, pe, ne, auxp, relg = _sc_gather_kernel()(
        h_s, p_s, n_s, perm, r_s, entity_user_embed, aux_pack,
        relation_embed)
    starts = r_s[0::BT]
    cnts = r_s[BT - 1::BT] - starts + 1
    sinfo = jnp.stack([starts, cnts]).astype(jnp.int32)
    rf = r_s.astype(jnp.float32).reshape(B, 1)
    res = _tc_call(sinfo, he, pe, ne, auxp, relg, rf,
                   trans_M, aux_W.T, aux_b.reshape(1, D))
    p = jnp.sum(res, axis=(0, 1))
    kg_loss = p[0] / B
    l2_loss = (p[1] + p[2] + p[3] + p[4]) / (2.0 * B)
    return kg_loss + KG_LAMBDA * l2_loss


# speedup vs baseline: 1.1360x; 1.0538x over previous
import functools

import jax
import jax.numpy as jnp
from jax import lax
from jax.experimental import pallas as pl
from jax.experimental.pallas import tpu as pltpu
from jax.experimental.pallas import tpu_sc as plsc

B = 16384
D = 64
N_REL = 64
N_AUX = 3
KG_LAMBDA = 1e-05
AUXW = 16

NC = 2
NS = 16
NW = NC * NS
BPW = B // NW
CH = 128
NCH = BPW // CH
CHE = 64
NCHE = BPW // CHE

BT = 512
G = B // BT


def _ent_gather_body(h_hbm, p_hbm, n_hbm, tab_hbm, out_h, out_p, out_n,
                     idx_v, rows_v, sem):
    c = lax.axis_index("c")
    s = lax.axis_index("s")
    wid = s * NC + c
    base = wid * BPW
    lanes = lax.iota(jnp.int32, 16)
    for src, dst in ((h_hbm, out_h), (p_hbm, out_p), (n_hbm, out_n)):
        pltpu.sync_copy(src.at[pl.ds(base, BPW)], idx_v)
        for ci in range(NCHE):
            def fire(g, carry, _ci=ci):
                vec = idx_v[pl.ds(g * 16, 16)]
                row0 = g * 16 - _ci * CHE
                for lane in range(16):
                    t = jnp.sum(jnp.where(lanes == lane, vec, 0))
                    start = (t >> 3) * 8
                    pltpu.async_copy(tab_hbm.at[pl.ds(start, 8)],
                                     rows_v.at[row0 + lane], sem)
                return carry

            lax.fori_loop(ci * (CHE // 16), (ci + 1) * (CHE // 16), fire, 0)

            def drain(i, carry):
                pltpu.make_async_copy(tab_hbm.at[pl.ds(0, 8)],
                                      rows_v.at[0], sem).wait()
                return carry

            lax.fori_loop(0, CHE, drain, 0)
            pltpu.sync_copy(rows_v, dst.at[pl.ds(base + ci * CHE, CHE)])


@functools.lru_cache(maxsize=1)
def _ent_gather_kernel():
    return functools.partial(
        pl.kernel,
        mesh=plsc.VectorSubcoreMesh(core_axis_name="c", subcore_axis_name="s"),
        out_type=(
            jax.ShapeDtypeStruct((B, 8, D), jnp.float32),
            jax.ShapeDtypeStruct((B, 8, D), jnp.float32),
            jax.ShapeDtypeStruct((B, 8, D), jnp.float32),
        ),
        scratch_types=[
            pltpu.VMEM((BPW,), jnp.int32),
            pltpu.VMEM((CHE, 8, D), jnp.float32),
            pltpu.SemaphoreType.DMA,
        ],
        compiler_params=pltpu.CompilerParams(needs_layout_passes=False),
    )(_ent_gather_body)


def _aux_gather_body(perm_hbm, r_hbm, aux_hbm, rel_hbm, out_a, out_r,
                     idx_v, aux_v, rel_v, sem):
    c = lax.axis_index("c")
    s = lax.axis_index("s")
    wid = s * NC + c
    base = wid * BPW
    for j in range(NCH):
        pltpu.sync_copy(r_hbm.at[pl.ds(base + j * CH, CH)], idx_v.at[j])
    handles = [
        pltpu.async_copy(rel_hbm.at[idx_v.at[j]],
                         rel_v.at[pl.ds(j * CH, CH)], sem)
        for j in range(NCH)
    ]
    for hd in handles:
        hd.wait()
    pltpu.sync_copy(rel_v, out_r.at[pl.ds(base, BPW)])
    for j in range(NCH):
        pltpu.sync_copy(perm_hbm.at[pl.ds(base + j * CH, CH)], idx_v.at[j])
    handles = [
        pltpu.async_copy(aux_hbm.at[idx_v.at[j]],
                         aux_v.at[pl.ds(j * CH, CH)], sem)
        for j in range(NCH)
    ]
    for hd in handles:
        hd.wait()
    pltpu.sync_copy(aux_v, out_a.at[pl.ds(base, BPW)])


@functools.lru_cache(maxsize=1)
def _aux_gather_kernel():
    return functools.partial(
        pl.kernel,
        mesh=plsc.VectorSubcoreMesh(core_axis_name="c", subcore_axis_name="s"),
        out_type=(
            jax.ShapeDtypeStruct((B, AUXW), jnp.float32),
            jax.ShapeDtypeStruct((B, D), jnp.float32),
        ),
        scratch_types=[
            pltpu.VMEM((NCH, CH), jnp.int32),
            pltpu.VMEM((BPW, AUXW), jnp.float32),
            pltpu.VMEM((BPW, D), jnp.float32),
            pltpu.SemaphoreType.DMA,
        ],
        compiler_params=pltpu.CompilerParams(use_tc_tiling_on_sc=False),
    )(_aux_gather_body)


def _tc_body(sref, he_ref, pe_ref, ne_ref, auxp_ref, relg_ref, rf_ref,
             sel_ref, m_ref, awt_ref, ab_ref, out_ref):
    g = pl.program_id(0)
    start = sref[0, g]
    cnt = sref[1, g]
    awt = awt_ref[...]
    ab = ab_ref[...]
    auxp = auxp_ref[...]
    sel = sel_ref[...]

    def pick(e_ref, i):
        e3 = e_ref[...]
        o = sel[:, i:i + 1]
        acc = e3[:, 0, :]
        for q in range(1, 8):
            acc = jnp.where(o == float(q), e3[:, q, :], acc)
        return acc

    def fuse(e, a):
        v = jnp.tanh(jnp.dot(a, awt, preferred_element_type=jnp.float32) + ab)
        return e * (1.0 + v)

    he = fuse(pick(he_ref, 0), auxp[:, 0:N_AUX])
    pe = fuse(pick(pe_ref, 1), auxp[:, N_AUX:2 * N_AUX])
    ne = fuse(pick(ne_ref, 2), auxp[:, 2 * N_AUX:3 * N_AUX])
    rr = rf_ref[...]
    e3 = jnp.concatenate([he, pe, ne], axis=0)
    rr3 = jnp.concatenate([rr, rr, rr], axis=0)

    def body(w, acc):
        k = start + w
        mk = m_ref[k]
        kf = lax.convert_element_type(k, jnp.float32)
        em = jnp.where(rr3 == kf, e3, 0.0)
        return acc + jnp.dot(em, mk, preferred_element_type=jnp.float32)

    z = jnp.zeros((3 * BT, D), jnp.float32)
    acc3 = lax.fori_loop(0, cnt, body, z)
    acc_h = acc3[0:BT, :]
    acc_p = acc3[BT:2 * BT, :]
    acc_n = acc3[2 * BT:3 * BT, :]

    re = relg_ref[...]
    dp = acc_h + re - acc_p
    dn = acc_h + re - acc_n
    pos = jnp.sum(dp * dp, axis=1, keepdims=True)
    neg = jnp.sum(dn * dn, axis=1, keepdims=True)
    x = neg - pos
    nls = jnp.maximum(-x, 0.0) + jnp.log(1.0 + jnp.exp(-jnp.abs(x)))
    rh2 = jnp.sum(acc_h * acc_h, axis=1, keepdims=True)
    re2 = jnp.sum(re * re, axis=1, keepdims=True)
    rp2 = jnp.sum(acc_p * acc_p, axis=1, keepdims=True)
    rn2 = jnp.sum(acc_n * acc_n, axis=1, keepdims=True)
    zz = jnp.zeros((BT, 3), jnp.float32)
    cat = jnp.concatenate([nls, rh2, re2, rp2, rn2, zz], axis=1)
    out_ref[...] = jnp.sum(cat, axis=0, keepdims=True).reshape(1, 1, 8)


def _tc_call(sinfo, he, pe, ne, auxp, relg, rf, sel, m, awt, ab):
    grid_spec = pltpu.PrefetchScalarGridSpec(
        num_scalar_prefetch=1,
        grid=(G,),
        in_specs=[
            pl.BlockSpec((BT, 8, D), lambda g, s: (g, 0, 0)),
            pl.BlockSpec((BT, 8, D), lambda g, s: (g, 0, 0)),
            pl.BlockSpec((BT, 8, D), lambda g, s: (g, 0, 0)),
            pl.BlockSpec((BT, AUXW), lambda g, s: (g, 0)),
            pl.BlockSpec((BT, D), lambda g, s: (g, 0)),
            pl.BlockSpec((BT, 1), lambda g, s: (g, 0)),
            pl.BlockSpec((BT, 3), lambda g, s: (g, 0)),
            pl.BlockSpec((N_REL, D, D), lambda g, s: (0, 0, 0)),
            pl.BlockSpec((N_AUX, D), lambda g, s: (0, 0)),
            pl.BlockSpec((1, D), lambda g, s: (0, 0)),
        ],
        out_specs=pl.BlockSpec((1, 1, 8), lambda g, s: (g, 0, 0)),
    )
    return pl.pallas_call(
        _tc_body,
        grid_spec=grid_spec,
        out_shape=jax.ShapeDtypeStruct((G, 1, 8), jnp.float32),
    )(sinfo, he, pe, ne, auxp, relg, rf, sel, m, awt, ab)


def kernel(h, r, pos_t, neg_t, h_aux, pos_t_aux, neg_t_aux,
           entity_user_embed, relation_embed, trans_M, aux_W, aux_b):
    h = h.astype(jnp.int32)
    r = r.astype(jnp.int32)
    pos_t = pos_t.astype(jnp.int32)
    neg_t = neg_t.astype(jnp.int32)
    perm = jnp.argsort(r).astype(jnp.int32)
    r_s = jnp.take(r, perm)
    h_s = jnp.take(h, perm)
    p_s = jnp.take(pos_t, perm)
    n_s = jnp.take(neg_t, perm)
    aux_pack = jnp.concatenate(
        [h_aux, pos_t_aux, neg_t_aux,
         jnp.zeros((B, AUXW - 3 * N_AUX), jnp.float32)], axis=1)
    sel = jnp.stack([(h_s & 7), (p_s & 7), (n_s & 7)],
                    axis=1).astype(jnp.float32)
    he, pe, ne = _ent_gather_kernel()(h_s, p_s, n_s, entity_user_embed)
    auxp, relg = _aux_gather_kernel()(perm, r_s, aux_pack, relation_embed)
    starts = r_s[0::BT]
    cnts = r_s[BT - 1::BT] - starts + 1
    sinfo = jnp.stack([starts, cnts]).astype(jnp.int32)
    rf = r_s.astype(jnp.float32).reshape(B, 1)
    res = _tc_call(sinfo, he, pe, ne, auxp, relg, rf, sel,
                   trans_M, aux_W.T, aux_b.reshape(1, D))
    p = jnp.sum(res, axis=(0, 1))
    kg_loss = p[0] / B
    l2_loss = (p[1] + p[2] + p[3] + p[4]) / (2.0 * B)
    return kg_loss + KG_LAMBDA * l2_loss


# speedup vs baseline: 1.6283x; 1.4333x over previous
import functools

import jax
import jax.numpy as jnp
from jax import lax
from jax.experimental import pallas as pl
from jax.experimental.pallas import tpu as pltpu
from jax.experimental.pallas import tpu_sc as plsc

B = 16384
D = 64
N_REL = 64
N_AUX = 3
KG_LAMBDA = 1e-05
AUXW = 16

NC = 2
NS = 16
NW = NC * NS
BPW = B // NW
CH = 128
NCH = BPW // CH
CHE = 64
NCHE = BPW // CHE

BT = 512
G = B // BT


def _ent_gather_body(h_hbm, p_hbm, n_hbm, tab_hbm, out_h, out_p, out_n,
                     idx_v, rows_v, sem):
    c = lax.axis_index("c")
    s = lax.axis_index("s")
    wid = s * NC + c
    base = wid * BPW
    lanes = lax.iota(jnp.int32, 16)
    for src, dst in ((h_hbm, out_h), (p_hbm, out_p), (n_hbm, out_n)):
        pltpu.sync_copy(src.at[pl.ds(base, BPW)], idx_v)
        for ci in range(NCHE):
            def fire(g, carry, _ci=ci):
                vec = idx_v[pl.ds(g * 16, 16)]
                row0 = g * 16 - _ci * CHE
                for lane in range(16):
                    t = jnp.sum(jnp.where(lanes == lane, vec, 0))
                    pltpu.async_copy(tab_hbm.at[pl.ds(t, 1)],
                                     rows_v.at[pl.ds(row0 + lane, 1)], sem)
                return carry

            lax.fori_loop(ci * (CHE // 16), (ci + 1) * (CHE // 16), fire, 0)

            def drain(i, carry):
                pltpu.make_async_copy(tab_hbm.at[pl.ds(0, 1)],
                                      rows_v.at[pl.ds(0, 1)], sem).wait()
                return carry

            lax.fori_loop(0, CHE, drain, 0)
            pltpu.sync_copy(rows_v, dst.at[pl.ds(base + ci * CHE, CHE)])


@functools.lru_cache(maxsize=1)
def _ent_gather_kernel():
    return functools.partial(
        pl.kernel,
        mesh=plsc.VectorSubcoreMesh(core_axis_name="c", subcore_axis_name="s"),
        out_type=(
            jax.ShapeDtypeStruct((B, D), jnp.float32),
            jax.ShapeDtypeStruct((B, D), jnp.float32),
            jax.ShapeDtypeStruct((B, D), jnp.float32),
        ),
        scratch_types=[
            pltpu.VMEM((BPW,), jnp.int32),
            pltpu.VMEM((CHE, D), jnp.float32),
            pltpu.SemaphoreType.DMA,
        ],
        compiler_params=pltpu.CompilerParams(needs_layout_passes=False),
    )(_ent_gather_body)


def _aux_gather_body(perm_hbm, r_hbm, aux_hbm, rel_hbm, out_a, out_r,
                     idx_v, aux_v, rel_v, sem):
    c = lax.axis_index("c")
    s = lax.axis_index("s")
    wid = s * NC + c
    base = wid * BPW
    for j in range(NCH):
        pltpu.sync_copy(r_hbm.at[pl.ds(base + j * CH, CH)], idx_v.at[j])
    handles = [
        pltpu.async_copy(rel_hbm.at[idx_v.at[j]],
                         rel_v.at[pl.ds(j * CH, CH)], sem)
        for j in range(NCH)
    ]
    for hd in handles:
        hd.wait()
    pltpu.sync_copy(rel_v, out_r.at[pl.ds(base, BPW)])
    for j in range(NCH):
        pltpu.sync_copy(perm_hbm.at[pl.ds(base + j * CH, CH)], idx_v.at[j])
    handles = [
        pltpu.async_copy(aux_hbm.at[idx_v.at[j]],
                         aux_v.at[pl.ds(j * CH, CH)], sem)
        for j in range(NCH)
    ]
    for hd in handles:
        hd.wait()
    pltpu.sync_copy(aux_v, out_a.at[pl.ds(base, BPW)])


@functools.lru_cache(maxsize=1)
def _aux_gather_kernel():
    return functools.partial(
        pl.kernel,
        mesh=plsc.VectorSubcoreMesh(core_axis_name="c", subcore_axis_name="s"),
        out_type=(
            jax.ShapeDtypeStruct((B, AUXW), jnp.float32),
            jax.ShapeDtypeStruct((B, D), jnp.float32),
        ),
        scratch_types=[
            pltpu.VMEM((NCH, CH), jnp.int32),
            pltpu.VMEM((BPW, AUXW), jnp.float32),
            pltpu.VMEM((BPW, D), jnp.float32),
            pltpu.SemaphoreType.DMA,
        ],
        compiler_params=pltpu.CompilerParams(use_tc_tiling_on_sc=False),
    )(_aux_gather_body)


def _tc_body(sref, he_ref, pe_ref, ne_ref, auxp_ref, relg_ref, rf_ref,
             m_ref, awt_ref, ab_ref, out_ref):
    g = pl.program_id(0)
    start = sref[0, g]
    cnt = sref[1, g]
    awt = awt_ref[...]
    ab = ab_ref[...]
    auxp = auxp_ref[...]
    def fuse(e_ref, a):
        v = jnp.tanh(jnp.dot(a, awt, preferred_element_type=jnp.float32) + ab)
        return e_ref[...] * (1.0 + v)

    he = fuse(he_ref, auxp[:, 0:N_AUX])
    pe = fuse(pe_ref, auxp[:, N_AUX:2 * N_AUX])
    ne = fuse(ne_ref, auxp[:, 2 * N_AUX:3 * N_AUX])
    rr = rf_ref[...]
    e3 = jnp.concatenate([he, pe, ne], axis=0)
    rr3 = jnp.concatenate([rr, rr, rr], axis=0)

    def body(w, acc):
        k = start + w
        mk = m_ref[k]
        kf = lax.convert_element_type(k, jnp.float32)
        em = jnp.where(rr3 == kf, e3, 0.0)
        return acc + jnp.dot(em, mk, preferred_element_type=jnp.float32)

    z = jnp.zeros((3 * BT, D), jnp.float32)
    acc3 = lax.fori_loop(0, cnt, body, z)
    acc_h = acc3[0:BT, :]
    acc_p = acc3[BT:2 * BT, :]
    acc_n = acc3[2 * BT:3 * BT, :]

    re = relg_ref[...]
    dp = acc_h + re - acc_p
    dn = acc_h + re - acc_n
    pos = jnp.sum(dp * dp, axis=1, keepdims=True)
    neg = jnp.sum(dn * dn, axis=1, keepdims=True)
    x = neg - pos
    nls = jnp.maximum(-x, 0.0) + jnp.log(1.0 + jnp.exp(-jnp.abs(x)))
    rh2 = jnp.sum(acc_h * acc_h, axis=1, keepdims=True)
    re2 = jnp.sum(re * re, axis=1, keepdims=True)
    rp2 = jnp.sum(acc_p * acc_p, axis=1, keepdims=True)
    rn2 = jnp.sum(acc_n * acc_n, axis=1, keepdims=True)
    zz = jnp.zeros((BT, 3), jnp.float32)
    cat = jnp.concatenate([nls, rh2, re2, rp2, rn2, zz], axis=1)
    out_ref[...] = jnp.sum(cat, axis=0, keepdims=True).reshape(1, 1, 8)


def _tc_call(sinfo, he, pe, ne, auxp, relg, rf, m, awt, ab):
    grid_spec = pltpu.PrefetchScalarGridSpec(
        num_scalar_prefetch=1,
        grid=(G,),
        in_specs=[
            pl.BlockSpec((BT, D), lambda g, s: (g, 0)),
            pl.BlockSpec((BT, D), lambda g, s: (g, 0)),
            pl.BlockSpec((BT, D), lambda g, s: (g, 0)),
            pl.BlockSpec((BT, AUXW), lambda g, s: (g, 0)),
            pl.BlockSpec((BT, D), lambda g, s: (g, 0)),
            pl.BlockSpec((BT, 1), lambda g, s: (g, 0)),
            pl.BlockSpec((N_REL, D, D), lambda g, s: (0, 0, 0)),
            pl.BlockSpec((N_AUX, D), lambda g, s: (0, 0)),
            pl.BlockSpec((1, D), lambda g, s: (0, 0)),
        ],
        out_specs=pl.BlockSpec((1, 1, 8), lambda g, s: (g, 0, 0)),
    )
    return pl.pallas_call(
        _tc_body,
        grid_spec=grid_spec,
        out_shape=jax.ShapeDtypeStruct((G, 1, 8), jnp.float32),
    )(sinfo, he, pe, ne, auxp, relg, rf, m, awt, ab)


def kernel(h, r, pos_t, neg_t, h_aux, pos_t_aux, neg_t_aux,
           entity_user_embed, relation_embed, trans_M, aux_W, aux_b):
    h = h.astype(jnp.int32)
    r = r.astype(jnp.int32)
    pos_t = pos_t.astype(jnp.int32)
    neg_t = neg_t.astype(jnp.int32)
    perm = jnp.argsort(r).astype(jnp.int32)
    r_s = jnp.take(r, perm)
    h_s = jnp.take(h, perm)
    p_s = jnp.take(pos_t, perm)
    n_s = jnp.take(neg_t, perm)
    aux_pack = jnp.concatenate(
        [h_aux, pos_t_aux, neg_t_aux,
         jnp.zeros((B, AUXW - 3 * N_AUX), jnp.float32)], axis=1)
    he, pe, ne = _ent_gather_kernel()(h_s, p_s, n_s, entity_user_embed)
    auxp, relg = _aux_gather_kernel()(perm, r_s, aux_pack, relation_embed)
    starts = r_s[0::BT]
    cnts = r_s[BT - 1::BT] - starts + 1
    sinfo = jnp.stack([starts, cnts]).astype(jnp.int32)
    rf = r_s.astype(jnp.float32).reshape(B, 1)
    res = _tc_call(sinfo, he, pe, ne, auxp, relg, rf,
                   trans_M, aux_W.T, aux_b.reshape(1, D))
    p = jnp.sum(res, axis=(0, 1))
    kg_loss = p[0] / B
    l2_loss = (p[1] + p[2] + p[3] + p[4]) / (2.0 * B)
    return kg_loss + KG_LAMBDA * l2_loss
